# baseline (speedup 1.0000x reference)
import functools

import jax
import jax.numpy as jnp
from jax import lax
from jax.experimental import pallas as pl
from jax.experimental.pallas import tpu as pltpu
from jax.experimental.pallas import tpu_sc as plsc

_N_EMB = 1024
_DIM = 64
_TOKENS = 16 * 1024
_BLK = 2048
_GRID = _TOKENS // _BLK
_LANES = 128
_NCHUNK = _N_EMB // _LANES

_NC = 2
_NS = 16
_NW = _NC * _NS
_ROWS_PER_W = _TOKENS // _NW
_STREAMS = 4
_ROWS_PER_S = _ROWS_PER_W // _STREAMS


def _vq_body(x_ref, e_ref, idx_ref, idxsc_ref, table_ref, losssum_ref):
    i = pl.program_id(0)
    x = x_ref[...].reshape(_BLK, _DIM)
    e = e_ref[...]
    x2 = jnp.sum(x * x, axis=1, keepdims=True)
    e2 = jnp.sum(e * e, axis=1)
    xm2 = x * (-2.0)
    xe = lax.dot_general(xm2, e, (((1,), (1,)), ((), ())),
                         preferred_element_type=jnp.float32)
    d = (x2 + e2[None, :]) + xe
    runm = d[:, 0:_LANES]
    runc = jnp.zeros((_BLK, _LANES), jnp.int32)
    for c in range(1, _NCHUNK):
        dc = d[:, c * _LANES:(c + 1) * _LANES]
        lt = dc < runm
        runm = jnp.where(lt, dc, runm)
        runc = jnp.where(lt, c, runc)
    m = jnp.min(runm, axis=1, keepdims=True)
    lane = lax.broadcasted_iota(jnp.int32, (_BLK, _LANES), 1)
    cand = jnp.where(runm == m, runc * _LANES + lane, _N_EMB)
    idx = jnp.min(cand, axis=1)
    idx_ref[...] = idx.reshape(_BLK, 1)
    idxsc_ref[...] = idx.reshape(1, 1, _BLK)

    @pl.when(i == 0)
    def _table():
        tbl = e.astype(jnp.bfloat16).astype(jnp.float32)
        table_ref[...] = jnp.concatenate(
            [tbl, jnp.zeros((_N_EMB, _LANES - _DIM), jnp.float32)], axis=1)
        losssum_ref[0, 0] = 0.0

    losssum_ref[0, 0] += jnp.sum(m)

    @pl.when(i == _GRID - 1)
    def _finalize():
        losssum_ref[0, 0] *= 1.0 / float(_TOKENS * _DIM)


_sc_mesh = plsc.VectorSubcoreMesh(core_axis_name="c", subcore_axis_name="s")


@functools.partial(
    pl.kernel,
    mesh=_sc_mesh,
    out_type=jax.ShapeDtypeStruct((_TOKENS, _LANES), jnp.float32),
    scratch_types=[
        pltpu.VMEM((_STREAMS, _ROWS_PER_S), jnp.int32),
        pltpu.VMEM((_ROWS_PER_W, _LANES), jnp.float32),
        pltpu.SemaphoreType.DMA,
    ],
)
def _sc_gather(table_hbm, idx_hbm, out_hbm, idx_v, rows_v, sem):
    wid = lax.axis_index("s") * _NC + lax.axis_index("c")
    blk = wid // (_BLK // _ROWS_PER_W)
    off = (wid % (_BLK // _ROWS_PER_W)) * _ROWS_PER_W
    for j in range(_STREAMS):
        pltpu.sync_copy(
            idx_hbm.at[blk, 0, pl.ds(off + j * _ROWS_PER_S, _ROWS_PER_S)],
            idx_v.at[j])
    copies = [
        pltpu.async_copy(table_hbm.at[idx_v.at[j]],
                         rows_v.at[pl.ds(j * _ROWS_PER_S, _ROWS_PER_S)], sem)
        for j in range(_STREAMS)
    ]
    for c in copies:
        c.wait()
    pltpu.sync_copy(rows_v, out_hbm.at[pl.ds(wid * _ROWS_PER_W, _ROWS_PER_W)])


@jax.jit
def kernel(inputs, embeddings):
    idx2, idxsc, table, losssum = pl.pallas_call(
        _vq_body,
        grid=(_GRID,),
        in_specs=[
            pl.BlockSpec((_BLK // 1024, 1024, _DIM), lambda i: (i, 0, 0)),
            pl.BlockSpec((_N_EMB, _DIM), lambda i: (0, 0)),
        ],
        out_specs=[
            pl.BlockSpec((_BLK, 1), lambda i: (i, 0)),
            pl.BlockSpec((1, 1, _BLK), lambda i: (i, 0, 0)),
            pl.BlockSpec((_N_EMB, _LANES), lambda i: (0, 0)),
            pl.BlockSpec(memory_space=pltpu.SMEM),
        ],
        out_shape=[
            jax.ShapeDtypeStruct((_TOKENS, 1), jnp.int32),
            jax.ShapeDtypeStruct((_GRID, 1, _BLK), jnp.int32),
            jax.ShapeDtypeStruct((_N_EMB, _LANES), jnp.float32),
            jax.ShapeDtypeStruct((1, 1), jnp.float32),
        ],
    )(inputs, embeddings)
    q = _sc_gather(table, idxsc)
    return q[:, :_DIM].reshape(inputs.shape), losssum[0, 0], idx2


# speedup vs baseline: 3.2322x; 3.2322x over previous
import jax
import jax.numpy as jnp
from jax import lax
from jax.experimental import pallas as pl
from jax.experimental.pallas import tpu as pltpu

_N_EMB = 1024
_DIM = 64
_BATCH = 16
_TOK = 1024
_TOKENS = _BATCH * _TOK
_LANES = 128
_NCHUNK = _N_EMB // _LANES


def _vq_body(xt_ref, et_ref, qt_ref, idx_ref, losssum_ref):
    i = pl.program_id(0)
    xt = xt_ref[...].reshape(_DIM, _TOK)
    et = et_ref[...]
    x2 = jnp.sum(xt * xt, axis=0, keepdims=True)
    e2 = jnp.sum(et * et, axis=0, keepdims=True)
    e2c = e2.reshape(_N_EMB, 1)
    xe = lax.dot_general(et, xt * (-2.0), (((0,), (0,)), ((), ())),
                         preferred_element_type=jnp.float32)
    d = (x2 + e2c) + xe
    runm = d[0:_LANES, :]
    runc = jnp.zeros((_LANES, _TOK), jnp.int32)
    for c in range(1, _NCHUNK):
        dc = d[c * _LANES:(c + 1) * _LANES, :]
        lt = dc < runm
        runm = jnp.where(lt, dc, runm)
        runc = jnp.where(lt, c, runc)
    m = jnp.min(runm, axis=0, keepdims=True)
    row = lax.broadcasted_iota(jnp.int32, (_LANES, _TOK), 0)
    cand = jnp.where(runm == m, runc * _LANES + row, _N_EMB)
    idx = jnp.min(cand, axis=0)
    idx_ref[...] = idx.reshape(_TOK // _LANES, _LANES)

    crow = lax.broadcasted_iota(jnp.int32, (_N_EMB, _TOK), 0)
    oht = (crow == idx[None, :]).astype(jnp.bfloat16)
    et_bf = et.astype(jnp.bfloat16)
    qt_ref[...] = lax.dot_general(
        et_bf, oht, (((1,), (0,)), ((), ())),
        preferred_element_type=jnp.float32).reshape(1, _DIM, _TOK)

    @pl.when(i == 0)
    def _init():
        losssum_ref[0, 0] = 0.0

    losssum_ref[0, 0] += jnp.sum(m)

    @pl.when(i == pl.num_programs(0) - 1)
    def _finalize():
        losssum_ref[0, 0] *= 1.0 / float(_TOKENS * _DIM)


@jax.jit
def kernel(inputs, embeddings):
    xt = inputs.transpose(0, 2, 1)
    et = embeddings.T
    qt, idx128, losssum = pl.pallas_call(
        _vq_body,
        grid=(_BATCH,),
        in_specs=[
            pl.BlockSpec((1, _DIM, _TOK), lambda i: (i, 0, 0)),
            pl.BlockSpec((_DIM, _N_EMB), lambda i: (0, 0)),
        ],
        out_specs=[
            pl.BlockSpec((1, _DIM, _TOK), lambda i: (i, 0, 0)),
            pl.BlockSpec((_TOK // _LANES, _LANES), lambda i: (i, 0)),
            pl.BlockSpec(memory_space=pltpu.SMEM),
        ],
        out_shape=[
            jax.ShapeDtypeStruct((_BATCH, _DIM, _TOK), jnp.float32),
            jax.ShapeDtypeStruct((_TOKENS // _LANES, _LANES), jnp.int32),
            jax.ShapeDtypeStruct((1, 1), jnp.float32),
        ],
    )(xt, et)
    q = qt.transpose(0, 2, 1)
    return q, losssum[0, 0], idx128.reshape(_TOKENS)[:, None]


# speedup vs baseline: 3.4300x; 1.0612x over previous
import jax
import jax.numpy as jnp
from jax import lax
from jax.experimental import pallas as pl
from jax.experimental.pallas import tpu as pltpu

_N_EMB = 1024
_DIM = 64
_BATCH = 16
_TOK = 1024
_SLABS = 2
_GRID = _BATCH // _SLABS
_TOKENS = _BATCH * _TOK
_LANES = 128
_NCHUNK = _N_EMB // _LANES


def _vq_body(xt_ref, et_ref, qt_ref, idx_ref, losssum_ref):
    i = pl.program_id(0)
    et = et_ref[...]
    e2 = jnp.sum(et * et, axis=0, keepdims=True)
    e2c = e2.reshape(_N_EMB, 1)
    et_bf = et.astype(jnp.bfloat16)
    msum = jnp.float32(0.0)
    for b in range(_SLABS):
        xt = xt_ref[b]
        x2 = jnp.sum(xt * xt, axis=0, keepdims=True)
        xe = lax.dot_general(et, xt * (-2.0), (((0,), (0,)), ((), ())),
                             preferred_element_type=jnp.float32)
        d = (x2 + e2c) + xe
        runm = d[0:_LANES, :]
        runc = jnp.zeros((_LANES, _TOK), jnp.int32)
        for c in range(1, _NCHUNK):
            dc = d[c * _LANES:(c + 1) * _LANES, :]
            lt = dc < runm
            runm = jnp.where(lt, dc, runm)
            runc = jnp.where(lt, c, runc)
        m = jnp.min(runm, axis=0, keepdims=True)
        row = lax.broadcasted_iota(jnp.int32, (_LANES, _TOK), 0)
        cand = jnp.where(runm == m, runc * _LANES + row, _N_EMB)
        idx = jnp.min(cand, axis=0)
        idx_ref[b * (_TOK // _LANES):(b + 1) * (_TOK // _LANES), :] = (
            idx.reshape(_TOK // _LANES, _LANES))

        crow = lax.broadcasted_iota(jnp.int32, (_N_EMB, _TOK), 0)
        oht = (crow == idx[None, :]).astype(jnp.bfloat16)
        qt_ref[b] = lax.dot_general(
            et_bf, oht, (((1,), (0,)), ((), ())),
            preferred_element_type=jnp.float32)
        msum += jnp.sum(m)

    @pl.when(i == 0)
    def _init():
        losssum_ref[0, 0] = 0.0

    losssum_ref[0, 0] += msum

    @pl.when(i == pl.num_programs(0) - 1)
    def _finalize():
        losssum_ref[0, 0] *= 1.0 / float(_TOKENS * _DIM)


@jax.jit
def kernel(inputs, embeddings):
    xt = inputs.transpose(0, 2, 1)
    et = embeddings.T
    qt, idx128, losssum = pl.pallas_call(
        _vq_body,
        grid=(_GRID,),
        in_specs=[
            pl.BlockSpec((_SLABS, _DIM, _TOK), lambda i: (i, 0, 0)),
            pl.BlockSpec((_DIM, _N_EMB), lambda i: (0, 0)),
        ],
        out_specs=[
            pl.BlockSpec((_SLABS, _DIM, _TOK), lambda i: (i, 0, 0)),
            pl.BlockSpec((_SLABS * _TOK // _LANES, _LANES), lambda i: (i, 0)),
            pl.BlockSpec(memory_space=pltpu.SMEM),
        ],
        out_shape=[
            jax.ShapeDtypeStruct((_BATCH, _DIM, _TOK), jnp.float32),
            jax.ShapeDtypeStruct((_TOKENS // _LANES, _LANES), jnp.int32),
            jax.ShapeDtypeStruct((1, 1), jnp.float32),
        ],
    )(xt, et)
    q = qt.transpose(0, 2, 1)
    return q, losssum[0, 0], idx128.reshape(_TOKENS)[:, None]


# speedup vs baseline: 3.5034x; 1.0214x over previous
import jax
import jax.numpy as jnp
from jax import lax
from jax.experimental import pallas as pl
from jax.experimental.pallas import tpu as pltpu

_N_EMB = 1024
_DIM = 64
_BATCH = 16
_TOK = 1024
_SLABS = 4
_GRID = _BATCH // _SLABS
_TOKENS = _BATCH * _TOK
_LANES = 128
_NCHUNK = _N_EMB // _LANES


def _vq_body(xt_ref, et_ref, qt_ref, idx_ref, losssum_ref):
    i = pl.program_id(0)
    et = et_ref[...]
    e2 = jnp.sum(et * et, axis=0, keepdims=True)
    e2c = e2.reshape(_N_EMB, 1)
    et_bf = et.astype(jnp.bfloat16)
    msum = jnp.float32(0.0)
    for b in range(_SLABS):
        xt = xt_ref[b]
        x2 = jnp.sum(xt * xt, axis=0, keepdims=True)
        xe = lax.dot_general(et, xt * (-2.0), (((0,), (0,)), ((), ())),
                             preferred_element_type=jnp.float32)
        d = (x2 + e2c) + xe
        runm = d[0:_LANES, :]
        runc = jnp.zeros((_LANES, _TOK), jnp.int32)
        for c in range(1, _NCHUNK):
            dc = d[c * _LANES:(c + 1) * _LANES, :]
            lt = dc < runm
            runm = jnp.where(lt, dc, runm)
            runc = jnp.where(lt, c, runc)
        m = jnp.min(runm, axis=0, keepdims=True)
        row = lax.broadcasted_iota(jnp.int32, (_LANES, _TOK), 0)
        cand = jnp.where(runm == m, runc * _LANES + row, _N_EMB)
        idx = jnp.min(cand, axis=0)
        idx_ref[b * (_TOK // _LANES):(b + 1) * (_TOK // _LANES), :] = (
            idx.reshape(_TOK // _LANES, _LANES))

        crow = lax.broadcasted_iota(jnp.int32, (_N_EMB, _TOK), 0)
        oht = (crow == idx[None, :]).astype(jnp.bfloat16)
        qt_ref[b] = lax.dot_general(
            et_bf, oht, (((1,), (0,)), ((), ())),
            preferred_element_type=jnp.float32)
        msum += jnp.sum(m)

    @pl.when(i == 0)
    def _init():
        losssum_ref[0, 0] = 0.0

    losssum_ref[0, 0] += msum

    @pl.when(i == pl.num_programs(0) - 1)
    def _finalize():
        losssum_ref[0, 0] *= 1.0 / float(_TOKENS * _DIM)


@jax.jit
def kernel(inputs, embeddings):
    xt = inputs.transpose(0, 2, 1)
    et = embeddings.T
    qt, idx128, losssum = pl.pallas_call(
        _vq_body,
        grid=(_GRID,),
        in_specs=[
            pl.BlockSpec((_SLABS, _DIM, _TOK), lambda i: (i, 0, 0)),
            pl.BlockSpec((_DIM, _N_EMB), lambda i: (0, 0)),
        ],
        out_specs=[
            pl.BlockSpec((_SLABS, _DIM, _TOK), lambda i: (i, 0, 0)),
            pl.BlockSpec((_SLABS * _TOK // _LANES, _LANES), lambda i: (i, 0)),
            pl.BlockSpec(memory_space=pltpu.SMEM),
        ],
        out_shape=[
            jax.ShapeDtypeStruct((_BATCH, _DIM, _TOK), jnp.float32),
            jax.ShapeDtypeStruct((_TOKENS // _LANES, _LANES), jnp.int32),
            jax.ShapeDtypeStruct((1, 1), jnp.float32),
        ],
    )(xt, et)
    q = qt.transpose(0, 2, 1)
    return q, losssum[0, 0], idx128.reshape(_TOKENS)[:, None]
